# Initial kernel scaffold; baseline (speedup 1.0000x reference)
import jax
import jax.numpy as jnp
from jax.experimental import pallas as pl


def kernel(h, edge_index, W_self1, W_neigh1, b1, W_self2, W_neigh2, b2):
    raise NotImplementedError("write your pallas kernel here")


# speedup vs baseline: 4.2505x; 4.2505x over previous
import functools

import jax
import jax.numpy as jnp
from jax import lax
from jax.experimental import pallas as pl
from jax.experimental.pallas import tpu as pltpu
from jax.experimental.pallas import tpu_sc as plsc

N = 10000
D = 128
E = 320000

NC = 2
NS = 16
NW = NC * NS
CH = 128
K = -(-E // (NW * CH))
EPW = K * CH
EPAD = EPW * NW
NPAD = 10240
ROWS_PT = NPAD // NS


def _sc_agg_body(x_hbm, srcw_hbm, dstw_hbm, zrow_hbm, zdeg_hbm,
                 agg_out, deg_out,
                 idxs_v, idxd_v, rows_v, ones_v, acc_sp, deg_sp, sem):
    c = lax.axis_index("c")
    s = lax.axis_index("s")
    wid = c * NS + s
    pltpu.sync_copy(zrow_hbm, acc_sp.at[pl.ds(s * ROWS_PT, ROWS_PT)])
    pltpu.sync_copy(zdeg_hbm, deg_sp.at[pl.ds(s * ROWS_PT, ROWS_PT)])
    pltpu.sync_copy(srcw_hbm.at[wid], idxs_v)
    pltpu.sync_copy(dstw_hbm.at[wid], idxd_v)
    for i in range(CH // 16):
        ones_v[pl.ds(i * 16, 16)] = jnp.ones((16,), jnp.float32)
    plsc.subcore_barrier()

    def chunk(j, carry):
        pltpu.async_copy(x_hbm.at[idxs_v.at[j]], rows_v, sem).wait()
        pltpu.sync_copy(rows_v, acc_sp.at[idxd_v.at[j]], add=True)
        pltpu.sync_copy(ones_v, deg_sp.at[idxd_v.at[j]], add=True)
        return carry

    lax.fori_loop(0, K, chunk, 0)
    plsc.subcore_barrier()
    pltpu.sync_copy(acc_sp.at[pl.ds(s * ROWS_PT, ROWS_PT)],
                    agg_out.at[c, pl.ds(s * ROWS_PT, ROWS_PT)])
    pltpu.sync_copy(deg_sp.at[pl.ds(s * ROWS_PT, ROWS_PT)],
                    deg_out.at[c, pl.ds(s * ROWS_PT, ROWS_PT)])


_sc_agg = functools.partial(
    pl.kernel,
    mesh=plsc.VectorSubcoreMesh(core_axis_name="c", subcore_axis_name="s"),
    out_type=[
        jax.ShapeDtypeStruct((NC, NPAD, D), jnp.float32),
        jax.ShapeDtypeStruct((NC, NPAD), jnp.float32),
    ],
    scratch_types=[
        pltpu.VMEM((K, CH), jnp.int32),
        pltpu.VMEM((K, CH), jnp.int32),
        pltpu.VMEM((CH, D), jnp.float32),
        pltpu.VMEM((CH,), jnp.float32),
        pltpu.VMEM_SHARED((NPAD, D), jnp.float32),
        pltpu.VMEM_SHARED((NPAD,), jnp.float32),
        pltpu.SemaphoreType.DMA,
    ],
)(_sc_agg_body)


def _layer_body(relu, h_ref, p0_ref, p1_ref, d0_ref, d1_ref,
                ws_ref, wn_ref, b_ref, o_ref):
    deg = jnp.maximum(d0_ref[...] + d1_ref[...], 1.0)
    neigh = (p0_ref[...] + p1_ref[...]) / deg
    acc = jnp.dot(h_ref[...], ws_ref[...], preferred_element_type=jnp.float32)
    acc += jnp.dot(neigh, wn_ref[...], preferred_element_type=jnp.float32)
    acc += b_ref[...]
    o_ref[...] = jnp.maximum(acc, 0.0) if relu else acc


def _tc_layer(h, p0, p1, d0, d1, Ws, Wn, b, relu):
    R = 400
    grid = (N // R,)
    row = pl.BlockSpec((R, D), lambda i: (i, 0))
    col = pl.BlockSpec((R, 1), lambda i: (i, 0))
    full = pl.BlockSpec((D, D), lambda i: (0, 0))
    bspec = pl.BlockSpec((1, D), lambda i: (0, 0))
    return pl.pallas_call(
        functools.partial(_layer_body, relu),
        grid=grid,
        in_specs=[row, row, row, col, col, full, full, bspec],
        out_specs=row,
        out_shape=jax.ShapeDtypeStruct((N, D), jnp.float32),
    )(h, p0, p1, d0, d1, Ws, Wn, b.reshape(1, D))


def kernel(h, edge_index, W_self1, W_neigh1, b1, W_self2, W_neigh2, b2):
    src = edge_index[0].astype(jnp.int32)
    dst = edge_index[1].astype(jnp.int32)
    pad = EPAD - E
    src_p = jnp.concatenate([src, jnp.zeros((pad,), jnp.int32)])
    dst_p = jnp.concatenate([dst, jnp.full((pad,), N, jnp.int32)])
    srcw = src_p.reshape(NW, K, CH)
    dstw = dst_p.reshape(NW, K, CH)
    zrow = jnp.zeros((ROWS_PT, D), jnp.float32)
    zdeg = jnp.zeros((ROWS_PT,), jnp.float32)

    aggp, degp = _sc_agg(h, srcw, dstw, zrow, zdeg)
    d0 = degp[0, :N, None]
    d1 = degp[1, :N, None]
    x = _tc_layer(h, aggp[0, :N], aggp[1, :N], d0, d1,
                  W_self1, W_neigh1, b1, True)
    aggp2, _ = _sc_agg(x, srcw, dstw, zrow, zdeg)
    out = _tc_layer(x, aggp2[0, :N], aggp2[1, :N], d0, d1,
                    W_self2, W_neigh2, b2, False)
    return out
